# baseline (speedup 1.0000x reference)
import functools

import jax
import jax.numpy as jnp
from jax import lax
from jax.experimental import pallas as pl
from jax.experimental.pallas import tpu as pltpu
from jax.experimental.pallas import tpu_sc as plsc

BATCH = 16384
DIM = 32
NC = 2
NS = 16
NW = NC * NS
BPW = BATCH // NW
CHUNK = 128
NCHUNK = BPW // CHUNK
GRP = 16


def _body(user_hbm, item_hbm, uidx_hbm, iidx_hbm, out_hbm,
          uidx_v, iidx_v, urows, irows, outv, sem_u, sem_i):
    wid = lax.axis_index("s") * NC + lax.axis_index("c")
    base = wid * BPW

    pltpu.sync_copy(uidx_hbm.at[pl.ds(wid * NCHUNK, NCHUNK)], uidx_v)
    pltpu.sync_copy(iidx_hbm.at[pl.ds(wid * NCHUNK, NCHUNK)], iidx_v)

    descs = []
    for j in range(NCHUNK):
        descs.append(pltpu.async_copy(
            user_hbm.at[uidx_v.at[j]], urows.at[pl.ds(j * CHUNK, CHUNK)],
            sem_u))
        descs.append(pltpu.async_copy(
            item_hbm.at[iidx_v.at[j]], irows.at[pl.ds(j * CHUNK, CHUNK)],
            sem_i))
    for dsc in descs:
        dsc.wait()

    def group(g, _):
        rows = jnp.full((GRP,), g * GRP, jnp.int32) + lax.iota(jnp.int32, GRP)
        acc = jnp.zeros((GRP,), jnp.float32)
        for d in range(DIM):
            col = jnp.full((GRP,), d, jnp.int32)
            uv = plsc.load_gather(urows, [rows, col])
            iv = plsc.load_gather(irows, [rows, col])
            acc = acc + uv * iv
        outv[pl.ds(pl.multiple_of(g * GRP, GRP), GRP)] = acc
        return 0

    lax.fori_loop(0, BPW // GRP, group, 0)

    pltpu.sync_copy(outv, out_hbm.at[pl.ds(base, BPW)])


@jax.jit
def _cf_dot(user_emb, item_emb, uidx, iidx):
    mesh = plsc.VectorSubcoreMesh(core_axis_name="c", subcore_axis_name="s")
    k = functools.partial(
        pl.kernel,
        mesh=mesh,
        out_type=jax.ShapeDtypeStruct((BATCH,), jnp.float32),
        scratch_types=[
            pltpu.VMEM((NCHUNK, CHUNK), jnp.int32),
            pltpu.VMEM((NCHUNK, CHUNK), jnp.int32),
            pltpu.VMEM((BPW, DIM), jnp.float32),
            pltpu.VMEM((BPW, DIM), jnp.float32),
            pltpu.VMEM((BPW,), jnp.float32),
            pltpu.SemaphoreType.DMA,
            pltpu.SemaphoreType.DMA,
        ],
        compiler_params=pltpu.CompilerParams(
            needs_layout_passes=False, use_tc_tiling_on_sc=False),
    )(_body)
    return k(user_emb, item_emb, uidx, iidx)


def kernel(x, user_emb, item_emb):
    x32 = x.astype(jnp.int32)
    uidx = x32[:, 0].reshape(BATCH // CHUNK, CHUNK)
    iidx = x32[:, 1].reshape(BATCH // CHUNK, CHUNK)
    return _cf_dot(user_emb, item_emb, uidx, iidx)
